# baseline (speedup 1.0000x reference)
import jax
import jax.numpy as jnp
from jax.experimental import pallas as pl
from jax.experimental.pallas import tpu as pltpu

N = 10000
NFEAT = 256
NHID = 128
BM = 400
G = N // BM
NBUF = 3
NPAIR = BM // 2
XCH = 2000
NXC = N // XCH


def _adj_copy(adj_ref, bufs, sems, blk, slot):
    return pltpu.make_async_copy(
        adj_ref.at[pl.ds(blk * BM, BM), :], bufs.at[slot], sems.at[slot])


def _x_copy(x_ref, xbufs, xsems, chunk, slot):
    return pltpu.make_async_copy(
        x_ref.at[pl.ds(chunk * XCH, XCH), :], xbufs.at[slot],
        xsems.at[slot])


def _main_kernel(x_ref, we_ref, adj_ref, b_ref, wd2_ref, bdec_ref,
                 lab_ref, out_ref, sup_ref, bufs, sems, xbufs, xsems):
    i = pl.program_id(0)

    @pl.when(i == 0)
    def _():
        _x_copy(x_ref, xbufs, xsems, 0, 0).start()
        _x_copy(x_ref, xbufs, xsems, 1, 1).start()
        for k in range(NBUF):
            _adj_copy(adj_ref, bufs, sems, k, k).start()
        we = we_ref[...].astype(jnp.bfloat16)
        for c in range(NXC):
            _x_copy(x_ref, xbufs, xsems, c, c % 2).wait()
            sup_ref[pl.ds(c * XCH, XCH), :] = jnp.dot(
                xbufs[c % 2].astype(jnp.bfloat16), we,
                preferred_element_type=jnp.float32).astype(jnp.bfloat16)
            if c + 2 < NXC:
                _x_copy(x_ref, xbufs, xsems, c + 2, c % 2).start()
        out_ref[0, 0] = 0.0

    slot = jax.lax.rem(i, NBUF)
    _adj_copy(adj_ref, bufs, sems, i, slot).wait()

    h = jnp.dot(bufs[slot].astype(jnp.bfloat16), sup_ref[...],
                preferred_element_type=jnp.float32)

    @pl.when(i + NBUF < G)
    def _():
        _adj_copy(adj_ref, bufs, sems, i + NBUF, slot).start()

    h = jnp.maximum(h + b_ref[...], 0.0)
    u = jnp.dot(h, wd2_ref[...], preferred_element_type=jnp.float32)
    row = jax.lax.broadcasted_iota(jnp.int32, (BM, 1), 0)
    w = jnp.where(row % 2 == 0, u[:, 0:1], u[:, 1:2])
    pr = jax.lax.broadcasted_iota(jnp.int32, (NPAIR, BM), 0)
    ci = jax.lax.broadcasted_iota(jnp.int32, (NPAIR, BM), 1)
    S = (ci // 2 == pr).astype(jnp.float32)
    logits = jnp.dot(S, w, preferred_element_type=jnp.float32) + bdec_ref[0]
    lab = lab_ref[0]
    pos = jnp.sum(jnp.maximum(logits, 0.0)
                  + jnp.log1p(jnp.exp(-jnp.abs(logits))))
    cross = jnp.dot(lab, logits, preferred_element_type=jnp.float32)[0, 0]
    out_ref[0, 0] += pos - cross


def kernel(x, adj, label, W_enc, b_enc, W_dec, b_dec):
    wd2 = W_dec.reshape(2, NHID).T
    b2 = b_enc.reshape(1, NHID)
    lab3 = label.reshape(G, 1, NPAIR)

    total = pl.pallas_call(
        _main_kernel,
        grid=(G,),
        in_specs=[
            pl.BlockSpec(memory_space=pl.ANY),
            pl.BlockSpec((NFEAT, NHID), lambda i: (0, 0)),
            pl.BlockSpec(memory_space=pl.ANY),
            pl.BlockSpec((1, NHID), lambda i: (0, 0)),
            pl.BlockSpec((NHID, 2), lambda i: (0, 0)),
            pl.BlockSpec(memory_space=pltpu.SMEM),
            pl.BlockSpec((1, 1, NPAIR), lambda i: (i, 0, 0)),
        ],
        out_specs=pl.BlockSpec(memory_space=pltpu.SMEM),
        out_shape=jax.ShapeDtypeStruct((1, 1), jnp.float32),
        scratch_shapes=[pltpu.VMEM((N, NHID), jnp.bfloat16),
                        pltpu.VMEM((NBUF, BM, N), jnp.float32),
                        pltpu.SemaphoreType.DMA((NBUF,)),
                        pltpu.VMEM((2, XCH, NFEAT), jnp.float32),
                        pltpu.SemaphoreType.DMA((2,))],
    )(x, W_enc, adj, b2, wd2, b_dec, lab3)

    return total[0, 0] / jnp.float32(N // 2)


# speedup vs baseline: 1.0575x; 1.0575x over previous
import jax
import jax.numpy as jnp
from jax.experimental import pallas as pl
from jax.experimental.pallas import tpu as pltpu

N = 10000
NFEAT = 256
NHID = 128
BM = 200
G = N // BM
NBUF = 3
NPAIR = BM // 2


def _copy(adj_ref, bufs, sems, blk, slot):
    return pltpu.make_async_copy(
        adj_ref.at[pl.ds(blk * BM, BM), :], bufs.at[slot], sems.at[slot])


def _main_kernel(x_ref, we_ref, adj_ref, b_ref, wd2_ref, bdec_ref,
                 lab_ref, out_ref, sup_ref, bufs, sems):
    i = pl.program_id(0)

    @pl.when(i == 0)
    def _():
        for k in range(NBUF):
            _copy(adj_ref, bufs, sems, k, k).start()
        sup_ref[...] = jnp.dot(x_ref[...].astype(jnp.bfloat16),
                               we_ref[...].astype(jnp.bfloat16),
                               preferred_element_type=jnp.float32
                               ).astype(jnp.bfloat16)
        out_ref[0, 0] = 0.0

    slot = jax.lax.rem(i, NBUF)
    _copy(adj_ref, bufs, sems, i, slot).wait()

    h = jnp.dot(bufs[slot].astype(jnp.bfloat16), sup_ref[...],
                preferred_element_type=jnp.float32)

    @pl.when(i + NBUF < G)
    def _():
        _copy(adj_ref, bufs, sems, i + NBUF, slot).start()

    h = jnp.maximum(h + b_ref[...], 0.0)
    u = jnp.dot(h, wd2_ref[...], preferred_element_type=jnp.float32)
    row = jax.lax.broadcasted_iota(jnp.int32, (BM, 1), 0)
    w = jnp.where(row % 2 == 0, u[:, 0:1], u[:, 1:2])
    pr = jax.lax.broadcasted_iota(jnp.int32, (NPAIR, BM), 0)
    ci = jax.lax.broadcasted_iota(jnp.int32, (NPAIR, BM), 1)
    S = (ci // 2 == pr).astype(jnp.float32)
    logits = jnp.dot(S, w, preferred_element_type=jnp.float32) + bdec_ref[0]
    lab = lab_ref[0]
    pos = jnp.sum(jnp.maximum(logits, 0.0)
                  + jnp.log1p(jnp.exp(-jnp.abs(logits))))
    cross = jnp.dot(lab, logits, preferred_element_type=jnp.float32)[0, 0]
    out_ref[0, 0] += pos - cross


def kernel(x, adj, label, W_enc, b_enc, W_dec, b_dec):
    wd2 = W_dec.reshape(2, NHID).T
    b2 = b_enc.reshape(1, NHID)
    lab3 = label.reshape(G, 1, NPAIR)

    total = pl.pallas_call(
        _main_kernel,
        grid=(G,),
        in_specs=[
            pl.BlockSpec((N, NFEAT), lambda i: (0, 0)),
            pl.BlockSpec((NFEAT, NHID), lambda i: (0, 0)),
            pl.BlockSpec(memory_space=pl.ANY),
            pl.BlockSpec((1, NHID), lambda i: (0, 0)),
            pl.BlockSpec((NHID, 2), lambda i: (0, 0)),
            pl.BlockSpec(memory_space=pltpu.SMEM),
            pl.BlockSpec((1, 1, NPAIR), lambda i: (i, 0, 0)),
        ],
        out_specs=pl.BlockSpec(memory_space=pltpu.SMEM),
        out_shape=jax.ShapeDtypeStruct((1, 1), jnp.float32),
        scratch_shapes=[pltpu.VMEM((N, NHID), jnp.bfloat16),
                        pltpu.VMEM((NBUF, BM, N), jnp.float32),
                        pltpu.SemaphoreType.DMA((NBUF,))],
    )(x, W_enc, adj, b2, wd2, b_dec, lab3)

    return total[0, 0] / jnp.float32(N // 2)
